# Initial kernel scaffold; baseline (speedup 1.0000x reference)
import functools

import jax
import jax.numpy as jnp
from jax import lax
from jax.experimental import pallas as pl
from jax.experimental.pallas import tpu as pltpu
from jax.experimental.pallas import tpu_sc as plsc

B = 128
T = 100001
O = 50000
F = 16

NC = 2
NS = 16
NW = NC * NS
L = 16

CH = 8
NCHUNKS = O // CH
CPW = -(-NCHUNKS // NW)
NVR = B // L


def _sc_body(tapeT_hbm, idx_hbm, w_hbm, bias_hbm, out_hbm,
             idx_v, w_v, bias_v, rows_v, out_v, sem):
    wid = lax.axis_index("s") * NC + lax.axis_index("c")

    def chunk_body(g, carry):
        c = wid + NW * g

        @pl.when(c < NCHUNKS)
        def _():
            pltpu.sync_copy(idx_hbm.at[c], idx_v)
            pltpu.sync_copy(w_hbm.at[c], w_v)
            pltpu.sync_copy(bias_hbm.at[pl.ds(c * CH, CH)], bias_v)
            pltpu.async_copy(tapeT_hbm.at[idx_v], rows_v, sem).wait()
            for j in range(CH):
                bj = plsc.load_gather(
                    bias_v, [jnp.full((L,), j, dtype=jnp.int32)])
                accs = [bj] * NVR
                for f in range(F):
                    e = j * F + f
                    wv = plsc.load_gather(
                        w_v, [jnp.full((L,), e, dtype=jnp.int32)])
                    for v in range(NVR):
                        r = rows_v[e, pl.ds(v * L, L)]
                        accs[v] = accs[v] + wv * r
                for v in range(NVR):
                    out_v[j, pl.ds(v * L, L)] = jnp.maximum(accs[v], 0.0)
            pltpu.sync_copy(out_v, out_hbm.at[pl.ds(c * CH, CH)])

        return carry

    lax.fori_loop(0, CPW, chunk_body, None)


@functools.partial(
    pl.kernel,
    mesh=plsc.VectorSubcoreMesh(core_axis_name="c", subcore_axis_name="s"),
    out_type=jax.ShapeDtypeStruct((O, B), jnp.float32),
    scratch_types=[
        pltpu.VMEM((CH * F,), jnp.int32),
        pltpu.VMEM((CH * F,), jnp.float32),
        pltpu.VMEM((CH,), jnp.float32),
        pltpu.VMEM((CH * F, B), jnp.float32),
        pltpu.VMEM((CH, B), jnp.float32),
        pltpu.SemaphoreType.DMA,
    ],
)
def _sc_kernel(tapeT_hbm, idx_hbm, w_hbm, bias_hbm, out_hbm,
               idx_v, w_v, bias_v, rows_v, out_v, sem):
    _sc_body(tapeT_hbm, idx_hbm, w_hbm, bias_hbm, out_hbm,
             idx_v, w_v, bias_v, rows_v, out_v, sem)


@jax.jit
def kernel(tape, weights, bias, input_indices, output_indices):
    tapeT = tape[:, :O].T
    idx = input_indices.reshape(NCHUNKS, CH * F).astype(jnp.int32)
    wts = weights.reshape(NCHUNKS, CH * F)
    outT = _sc_kernel(tapeT, idx, wts, bias)
    return jnp.concatenate([tape[:, :O + 1], outT.T], axis=1)


# speedup vs baseline: 2.1417x; 2.1417x over previous
import functools

import jax
import jax.numpy as jnp
from jax import lax
from jax.experimental import pallas as pl
from jax.experimental.pallas import tpu as pltpu
from jax.experimental.pallas import tpu_sc as plsc

B = 128
T = 100001
O = 50000
F = 16

NC = 2
NS = 16
NW = NC * NS
L = 16

CH = 8
NCHUNKS = O // CH
CPW = -(-NCHUNKS // NW)
NVR = B // L


_GATHER_DNUMS = lax.GatherDimensionNumbers(
    offset_dims=(), collapsed_slice_dims=(0,), start_index_map=(0,))


def _lane_bcast(vec, lane):
    idx = jnp.full((L, 1), lane, dtype=jnp.int32)
    return lax.gather(vec, idx, _GATHER_DNUMS, (1,),
                      mode=lax.GatherScatterMode.PROMISE_IN_BOUNDS)


def _sc_body(tapeT_hbm, idx_hbm, w_hbm, bias_hbm, out_hbm,
             idx_v, w_v, bias_v, rows_v, out_v, sem):
    wid = lax.axis_index("s") * NC + lax.axis_index("c")

    def chunk_body(g, carry):
        c = wid + NW * g

        @pl.when(c < NCHUNKS)
        def _():
            pltpu.sync_copy(idx_hbm.at[c], idx_v)
            pltpu.sync_copy(w_hbm.at[c], w_v)
            pltpu.sync_copy(bias_hbm.at[pl.ds(c * CH, CH)],
                            bias_v.at[pl.ds(0, CH)])
            pltpu.async_copy(tapeT_hbm.at[idx_v], rows_v, sem).wait()
            brow = bias_v[pl.ds(0, L)]
            for j in range(CH):
                bj = _lane_bcast(brow, j)
                wrow = w_v[pl.ds(j * F, F)]
                accs = [bj] * NVR
                for f in range(F):
                    e = j * F + f
                    wv = _lane_bcast(wrow, f)
                    for v in range(NVR):
                        r = rows_v[e, pl.ds(v * L, L)]
                        accs[v] = accs[v] + wv * r
                for v in range(NVR):
                    out_v[j, pl.ds(v * L, L)] = jnp.maximum(accs[v], 0.0)
            pltpu.sync_copy(out_v, out_hbm.at[pl.ds(c * CH, CH)])

        return carry

    lax.fori_loop(0, CPW, chunk_body, None)


@functools.partial(
    pl.kernel,
    mesh=plsc.VectorSubcoreMesh(core_axis_name="c", subcore_axis_name="s"),
    out_type=jax.ShapeDtypeStruct((O, B), jnp.float32),
    scratch_types=[
        pltpu.VMEM((CH * F,), jnp.int32),
        pltpu.VMEM((CH * F,), jnp.float32),
        pltpu.VMEM((CH * F,), jnp.float32),
        pltpu.VMEM((CH * F, B), jnp.float32),
        pltpu.VMEM((CH, B), jnp.float32),
        pltpu.SemaphoreType.DMA,
    ],
)
def _sc_kernel(tapeT_hbm, idx_hbm, w_hbm, bias_hbm, out_hbm,
               idx_v, w_v, bias_v, rows_v, out_v, sem):
    _sc_body(tapeT_hbm, idx_hbm, w_hbm, bias_hbm, out_hbm,
             idx_v, w_v, bias_v, rows_v, out_v, sem)


@jax.jit
def kernel(tape, weights, bias, input_indices, output_indices):
    tapeT = tape[:, :O].T
    idx = input_indices.reshape(NCHUNKS, CH * F).astype(jnp.int32)
    wts = weights.reshape(NCHUNKS, CH * F)
    outT = _sc_kernel(tapeT, idx, wts, bias)
    return jnp.concatenate([tape[:, :O + 1], outT.T], axis=1)
